# baseline (speedup 1.0000x reference)
import functools

import jax
import jax.numpy as jnp
from jax import lax
from jax.experimental import pallas as pl
from jax.experimental.pallas import tpu as pltpu
from jax.experimental.pallas import tpu_sc as plsc

EPS = 1e-12
NC = 2
NS = 16
NW = NC * NS


def _sc_gather(table, ids, n_tokens, width, dtype):
    per_w = n_tokens // NW
    NBUF = 4
    AHEAD = 2
    CH = 64 if (per_w % 256 == 0 and per_w // 64 >= 12) else 32
    nch = per_w // CH
    ids3 = ids.reshape(NW, nch, CH)

    mesh = plsc.VectorSubcoreMesh(core_axis_name="c", subcore_axis_name="s")

    @functools.partial(
        pl.kernel,
        mesh=mesh,
        out_type=jax.ShapeDtypeStruct((n_tokens, width), dtype),
        scratch_types=[
            pltpu.VMEM((nch, CH), jnp.int32),
            [pltpu.VMEM((CH, width), dtype)] * NBUF,
            [pltpu.SemaphoreType.DMA] * NBUF,
            [pltpu.SemaphoreType.DMA] * NBUF,
        ],
    )
    def gather_k(table_hbm, ids_hbm, out_hbm, idx_v, rows, gsems, psems):
        wid = lax.axis_index("s") * NC + lax.axis_index("c")
        base = wid * per_w

        def gstart(c, b):
            pltpu.make_async_copy(table_hbm.at[idx_v.at[c]], rows[b], gsems[b]).start()

        def gwait(b):
            pltpu.make_async_copy(table_hbm.at[idx_v.at[0]], rows[b], gsems[b]).wait()

        def pstart(c, b):
            row_start = pl.multiple_of(base + c * CH, CH)
            pltpu.make_async_copy(rows[b], out_hbm.at[pl.ds(row_start, CH)], psems[b]).start()

        def pwait(b):
            pltpu.make_async_copy(rows[b], out_hbm.at[pl.ds(base, CH)], psems[b]).wait()

        pltpu.sync_copy(ids_hbm.at[wid], idx_v)
        for c0 in range(AHEAD):
            gstart(c0, c0 % NBUF)

        ngrp = nch // NBUF

        def group(i, first=False, last=False):
            for b in range(NBUF):
                c = i * NBUF + b
                gwait(b)
                pstart(c, b)
                if (not last) or (b < AHEAD):
                    bn = (b + AHEAD) % NBUF
                    if not (first and b < AHEAD):
                        pwait(bn)
                    gstart(c + AHEAD, bn)

        group(0, first=True)

        def body(i, _):
            group(i)
            return 0

        lax.fori_loop(1, ngrp - 1, body, 0)
        group(ngrp - 1, last=True)
        for b in range(NBUF):
            pwait(b)

    return gather_k(table, ids3)


def _tc_fused_slice(we3, token_type_ids, am, pe2, tokd2, gamma2, beta2,
                    prev_out, prev_mask, slice_idx, b_total, hd, b0, packed):
    Bs, S, W = we3.shape
    Hh = hd // 2
    BB = 16
    grid = (Bs // BB,)
    off = b0 // BB

    def _ln_store(streams, tt, td_ref, g_ref, b_ref, out_ref):
        embs = []
        s = 0.0
        sq = 0.0
        for x, lo, w in streams:
            emb = x + tt * td_ref[0][None, None, lo:lo + w]
            embs.append((emb, lo, w))
            s = s + jnp.sum(emb, axis=-1, keepdims=True)
            sq = sq + jnp.sum(emb * emb, axis=-1, keepdims=True)
        mu = s * (1.0 / hd)
        var = sq * (1.0 / hd) - mu * mu
        rstd = lax.rsqrt(var + EPS)
        mr = mu * rstd
        for emb, lo, w in embs:
            g = g_ref[0][None, None, lo:lo + w]
            b = b_ref[0][None, None, lo:lo + w]
            out_ref[:, :, lo:lo + w] = (emb * rstd - mr) * g + b

    def body(*refs):
        we_ref, tt_ref, am_ref, pe_ref, td_ref, g_ref, b_ref = refs[:7]
        out_ref, mask_ref = refs[-2], refs[-1]
        tt = tt_ref[...].astype(jnp.float32)[..., None]
        if packed:
            x = we_ref[...]
            lo = lax.bitcast_convert_type(x << 16, jnp.float32) + pe_ref[:, :Hh][None]
            hi = (lax.bitcast_convert_type(x & jnp.int32(-65536), jnp.float32)
                  + pe_ref[:, Hh:][None])
            streams = [(lo, 0, Hh), (hi, Hh, Hh)]
        else:
            streams = [(we_ref[...] + pe_ref[...][None], 0, hd)]
        _ln_store(streams, tt, td_ref, g_ref, b_ref, out_ref)
        amf = am_ref[...].astype(jnp.float32)
        mask_ref[...] = ((1.0 - amf) * -10000.0)[:, None, :]

    in_specs = [
        pl.BlockSpec((BB, S, W), lambda i: (i, 0, 0)),
        pl.BlockSpec((BB, S), lambda i: (i, 0)),
        pl.BlockSpec((BB, S), lambda i: (i, 0)),
        pl.BlockSpec((S, hd), lambda i: (0, 0)),
        pl.BlockSpec((1, hd), lambda i: (0, 0)),
        pl.BlockSpec((1, hd), lambda i: (0, 0)),
        pl.BlockSpec((1, hd), lambda i: (0, 0)),
    ]
    args = [we3, token_type_ids, am, pe2, tokd2, gamma2, beta2]
    aliases = {}
    if slice_idx > 0:
        in_specs.append(pl.BlockSpec(memory_space=pl.ANY))
        in_specs.append(pl.BlockSpec(memory_space=pl.ANY))
        args.append(prev_out)
        args.append(prev_mask)
        aliases = {7: 0, 8: 1}

    return pl.pallas_call(
        body,
        grid=grid,
        in_specs=in_specs,
        out_specs=[
            pl.BlockSpec((BB, S, hd), lambda i: (i + off, 0, 0)),
            pl.BlockSpec((BB, 1, S), lambda i: (i + off, 0, 0)),
        ],
        out_shape=[
            jax.ShapeDtypeStruct((b_total, S, hd), jnp.float32),
            jax.ShapeDtypeStruct((b_total, 1, S), jnp.float32),
        ],
        input_output_aliases=aliases,
    )(*args)


def _pack_table(word_emb):
    V, Hd = word_emb.shape
    Hh = Hd // 2
    R = 512
    grid = (pl.cdiv(V, R),)

    def body(w_ref, o_ref):
        x = w_ref[...]
        lob = x[:, :Hh].astype(jnp.bfloat16)
        hib = x[:, Hh:].astype(jnp.bfloat16)
        lo16 = lax.bitcast_convert_type(lob, jnp.uint16).astype(jnp.int32)
        hi16 = lax.bitcast_convert_type(hib, jnp.uint16).astype(jnp.int32)
        o_ref[...] = lo16 | (hi16 << 16)

    return pl.pallas_call(
        body,
        grid=grid,
        in_specs=[pl.BlockSpec((R, Hd), lambda i: (i, 0))],
        out_specs=pl.BlockSpec((R, Hh), lambda i: (i, 0)),
        out_shape=jax.ShapeDtypeStruct((V, Hh), jnp.int32),
    )(word_emb)


def kernel(input_ids, attention_mask, token_type_ids, word_emb, pos_emb, tok_emb, gamma, beta):
    B, S = input_ids.shape
    V, Hd = word_emb.shape
    n = B * S
    ids = input_ids.reshape(-1).astype(jnp.int32)
    SLICES = [96, 288, 320, 320]
    tt = token_type_ids.astype(jnp.int32)
    pe2 = pos_emb[:S] + tok_emb[0][None, :]
    tokd2 = (tok_emb[1] - tok_emb[0]).reshape(1, Hd)
    gamma2 = gamma.reshape(1, Hd)
    beta2 = beta.reshape(1, Hd)
    Hh = Hd // 2
    offs = [sum(SLICES[:i]) for i in range(len(SLICES))]
    we0 = (_sc_gather(word_emb, ids[:SLICES[0] * S], SLICES[0] * S, Hd, jnp.float32)
           .reshape(SLICES[0], S, Hd))
    word_emb_h = _pack_table(word_emb)
    we_slices = [we0] + [
        _sc_gather(word_emb_h, ids[b0 * S:(b0 + bs) * S], bs * S, Hh, jnp.int32)
        .reshape(bs, S, Hh)
        for b0, bs in zip(offs[1:], SLICES[1:])
    ]
    am = attention_mask.astype(jnp.int32)
    out = None
    mask = None
    for i, (b0, bs, we) in enumerate(zip(offs, SLICES, we_slices)):
        out, mask = _tc_fused_slice(we,
                                    tt[b0:b0 + bs],
                                    am[b0:b0 + bs],
                                    pe2, tokd2, gamma2, beta2,
                                    out, mask, i, B, Hd, b0, packed=(i > 0))
    return (out, mask)


# speedup vs baseline: 1.0204x; 1.0204x over previous
import functools

import jax
import jax.numpy as jnp
from jax import lax
from jax.experimental import pallas as pl
from jax.experimental.pallas import tpu as pltpu
from jax.experimental.pallas import tpu_sc as plsc

EPS = 1e-12
NC = 2
NS = 16
NW = NC * NS


def _sc_gather(table, ids, n_tokens, width, dtype):
    per_w = n_tokens // NW
    NBUF = 4
    AHEAD = 2
    CH = 64 if (per_w % 256 == 0 and per_w // 64 >= 12) else 32
    nch = per_w // CH
    ids3 = ids.reshape(NW, nch, CH)

    mesh = plsc.VectorSubcoreMesh(core_axis_name="c", subcore_axis_name="s")

    @functools.partial(
        pl.kernel,
        mesh=mesh,
        out_type=jax.ShapeDtypeStruct((n_tokens, width), dtype),
        scratch_types=[
            pltpu.VMEM((nch, CH), jnp.int32),
            [pltpu.VMEM((CH, width), dtype)] * NBUF,
            [pltpu.SemaphoreType.DMA] * NBUF,
            [pltpu.SemaphoreType.DMA] * NBUF,
        ],
    )
    def gather_k(table_hbm, ids_hbm, out_hbm, idx_v, rows, gsems, psems):
        wid = lax.axis_index("s") * NC + lax.axis_index("c")
        base = wid * per_w

        def gstart(c, b):
            pltpu.make_async_copy(table_hbm.at[idx_v.at[c]], rows[b], gsems[b]).start()

        def gwait(b):
            pltpu.make_async_copy(table_hbm.at[idx_v.at[0]], rows[b], gsems[b]).wait()

        def pstart(c, b):
            row_start = pl.multiple_of(base + c * CH, CH)
            pltpu.make_async_copy(rows[b], out_hbm.at[pl.ds(row_start, CH)], psems[b]).start()

        def pwait(b):
            pltpu.make_async_copy(rows[b], out_hbm.at[pl.ds(base, CH)], psems[b]).wait()

        pltpu.sync_copy(ids_hbm.at[wid], idx_v)
        for c0 in range(AHEAD):
            gstart(c0, c0 % NBUF)

        ngrp = nch // NBUF

        def group(i, first=False, last=False):
            for b in range(NBUF):
                c = i * NBUF + b
                gwait(b)
                pstart(c, b)
                if (not last) or (b < AHEAD):
                    bn = (b + AHEAD) % NBUF
                    if not (first and b < AHEAD):
                        pwait(bn)
                    gstart(c + AHEAD, bn)

        group(0, first=True)

        def body(i, _):
            group(i)
            return 0

        lax.fori_loop(1, ngrp - 1, body, 0)
        group(ngrp - 1, last=True)
        for b in range(NBUF):
            pwait(b)

    return gather_k(table, ids3)


def _tc_fused_slice(we3, token_type_ids, am, pe2, tokd2, gamma2, beta2,
                    prev_out, prev_mask, slice_idx, b_total, hd, b0, packed):
    Bs, S, W = we3.shape
    Hh = hd // 2
    BB = 16
    grid = (Bs // BB,)
    off = b0 // BB

    def _ln_store(streams, tt, td_ref, g_ref, b_ref, out_ref):
        embs = []
        s = 0.0
        sq = 0.0
        for x, lo, w in streams:
            emb = x + tt * td_ref[0][None, None, lo:lo + w]
            embs.append((emb, lo, w))
            s = s + jnp.sum(emb, axis=-1, keepdims=True)
            sq = sq + jnp.sum(emb * emb, axis=-1, keepdims=True)
        mu = s * (1.0 / hd)
        var = sq * (1.0 / hd) - mu * mu
        rstd = lax.rsqrt(var + EPS)
        mr = mu * rstd
        for emb, lo, w in embs:
            g = g_ref[0][None, None, lo:lo + w]
            b = b_ref[0][None, None, lo:lo + w]
            out_ref[:, :, lo:lo + w] = (emb * rstd - mr) * g + b

    def body(*refs):
        we_ref, tt_ref, am_ref, pe_ref, td_ref, g_ref, b_ref = refs[:7]
        out_ref, mask_ref = refs[-2], refs[-1]
        tt = tt_ref[...].astype(jnp.float32)[..., None]
        if packed:
            x = we_ref[...]
            lo = lax.bitcast_convert_type(x << 16, jnp.float32) + pe_ref[:, :Hh][None]
            hi = (lax.bitcast_convert_type(x & jnp.int32(-65536), jnp.float32)
                  + pe_ref[:, Hh:][None])
            streams = [(lo, 0, Hh), (hi, Hh, Hh)]
        else:
            streams = [(we_ref[...] + pe_ref[...][None], 0, hd)]
        _ln_store(streams, tt, td_ref, g_ref, b_ref, out_ref)
        amf = am_ref[...].astype(jnp.float32)
        mask_ref[...] = ((1.0 - amf) * -10000.0)[:, None, :]

    in_specs = [
        pl.BlockSpec((BB, S, W), lambda i: (i, 0, 0)),
        pl.BlockSpec((BB, S), lambda i: (i, 0)),
        pl.BlockSpec((BB, S), lambda i: (i, 0)),
        pl.BlockSpec((S, hd), lambda i: (0, 0)),
        pl.BlockSpec((1, hd), lambda i: (0, 0)),
        pl.BlockSpec((1, hd), lambda i: (0, 0)),
        pl.BlockSpec((1, hd), lambda i: (0, 0)),
    ]
    args = [we3, token_type_ids, am, pe2, tokd2, gamma2, beta2]
    aliases = {}
    if slice_idx > 0:
        in_specs.append(pl.BlockSpec(memory_space=pl.ANY))
        in_specs.append(pl.BlockSpec(memory_space=pl.ANY))
        args.append(prev_out)
        args.append(prev_mask)
        aliases = {7: 0, 8: 1}

    return pl.pallas_call(
        body,
        grid=grid,
        in_specs=in_specs,
        out_specs=[
            pl.BlockSpec((BB, S, hd), lambda i: (i + off, 0, 0)),
            pl.BlockSpec((BB, 1, S), lambda i: (i + off, 0, 0)),
        ],
        out_shape=[
            jax.ShapeDtypeStruct((b_total, S, hd), jnp.float32),
            jax.ShapeDtypeStruct((b_total, 1, S), jnp.float32),
        ],
        input_output_aliases=aliases,
    )(*args)


def _pack_table(word_emb):
    V, Hd = word_emb.shape
    Hh = Hd // 2
    R = 512
    grid = (pl.cdiv(V, R),)

    def body(w_ref, o_ref):
        x = w_ref[...]
        lob = x[:, :Hh].astype(jnp.bfloat16)
        hib = x[:, Hh:].astype(jnp.bfloat16)
        lo16 = lax.bitcast_convert_type(lob, jnp.uint16).astype(jnp.int32)
        hi16 = lax.bitcast_convert_type(hib, jnp.uint16).astype(jnp.int32)
        o_ref[...] = lo16 | (hi16 << 16)

    return pl.pallas_call(
        body,
        grid=grid,
        in_specs=[pl.BlockSpec((R, Hd), lambda i: (i, 0))],
        out_specs=pl.BlockSpec((R, Hh), lambda i: (i, 0)),
        out_shape=jax.ShapeDtypeStruct((V, Hh), jnp.int32),
    )(word_emb)


def kernel(input_ids, attention_mask, token_type_ids, word_emb, pos_emb, tok_emb, gamma, beta):
    B, S = input_ids.shape
    V, Hd = word_emb.shape
    n = B * S
    ids = input_ids.reshape(-1).astype(jnp.int32)
    SLICES = [96, 288, 320, 320]
    tt = token_type_ids.astype(jnp.int32)
    pe2 = pos_emb[:S] + tok_emb[0][None, :]
    tokd2 = (tok_emb[1] - tok_emb[0]).reshape(1, Hd)
    gamma2 = gamma.reshape(1, Hd)
    beta2 = beta.reshape(1, Hd)
    Hh = Hd // 2
    offs = [sum(SLICES[:i]) for i in range(len(SLICES))]
    word_emb_h = _pack_table(word_emb)
    we_slices = [
        _sc_gather(word_emb_h, ids[b0 * S:(b0 + bs) * S], bs * S, Hh, jnp.int32)
        .reshape(bs, S, Hh)
        for b0, bs in zip(offs, SLICES)
    ]
    am = attention_mask.astype(jnp.int32)
    out = None
    mask = None
    for i, (b0, bs, we) in enumerate(zip(offs, SLICES, we_slices)):
        out, mask = _tc_fused_slice(we,
                                    tt[b0:b0 + bs],
                                    am[b0:b0 + bs],
                                    pe2, tokd2, gamma2, beta2,
                                    out, mask, i, B, Hd, b0, packed=True)
    return (out, mask)


# speedup vs baseline: 1.0742x; 1.0527x over previous
import functools

import jax
import jax.numpy as jnp
from jax import lax
from jax.experimental import pallas as pl
from jax.experimental.pallas import tpu as pltpu
from jax.experimental.pallas import tpu_sc as plsc

EPS = 1e-12
NC = 2
NS = 16
NW = NC * NS


def _sc_gather(table, ids, n_tokens, width, dtype):
    per_w = n_tokens // NW
    NBUF = 4
    AHEAD = 2
    CH = 64 if (per_w % 256 == 0 and per_w // 64 >= 12) else 32
    nch = per_w // CH
    ids3 = ids.reshape(NW, nch, CH)

    mesh = plsc.VectorSubcoreMesh(core_axis_name="c", subcore_axis_name="s")

    @functools.partial(
        pl.kernel,
        mesh=mesh,
        out_type=jax.ShapeDtypeStruct((n_tokens, width), dtype),
        scratch_types=[
            pltpu.VMEM((nch, CH), jnp.int32),
            [pltpu.VMEM((CH, width), dtype)] * NBUF,
            [pltpu.SemaphoreType.DMA] * NBUF,
            [pltpu.SemaphoreType.DMA] * NBUF,
        ],
    )
    def gather_k(table_hbm, ids_hbm, out_hbm, idx_v, rows, gsems, psems):
        wid = lax.axis_index("s") * NC + lax.axis_index("c")
        base = wid * per_w

        def gstart(c, b):
            pltpu.make_async_copy(table_hbm.at[idx_v.at[c]], rows[b], gsems[b]).start()

        def gwait(b):
            pltpu.make_async_copy(table_hbm.at[idx_v.at[0]], rows[b], gsems[b]).wait()

        def pstart(c, b):
            row_start = pl.multiple_of(base + c * CH, CH)
            pltpu.make_async_copy(rows[b], out_hbm.at[pl.ds(row_start, CH)], psems[b]).start()

        def pwait(b):
            pltpu.make_async_copy(rows[b], out_hbm.at[pl.ds(base, CH)], psems[b]).wait()

        pltpu.sync_copy(ids_hbm.at[wid], idx_v)
        for c0 in range(AHEAD):
            gstart(c0, c0 % NBUF)

        ngrp = nch // NBUF

        def group(i, first=False, last=False):
            for b in range(NBUF):
                c = i * NBUF + b
                gwait(b)
                pstart(c, b)
                if (not last) or (b < AHEAD):
                    bn = (b + AHEAD) % NBUF
                    if not (first and b < AHEAD):
                        pwait(bn)
                    gstart(c + AHEAD, bn)

        group(0, first=True)

        def body(i, _):
            group(i)
            return 0

        lax.fori_loop(1, ngrp - 1, body, 0)
        group(ngrp - 1, last=True)
        for b in range(NBUF):
            pwait(b)

    return gather_k(table, ids3)


def _tc_fused_slice(we3, token_type_ids, am, pe2, tokd2, gamma2, beta2,
                    prev_out, prev_mask, slice_idx, b_total, hd, b0, packed):
    Bs, S, W = we3.shape
    Hh = hd // 2
    BB = 16
    grid = (Bs // BB,)
    off = b0 // BB

    def _ln_store(streams, tt, td_ref, g_ref, b_ref, out_ref):
        embs = []
        s = 0.0
        sq = 0.0
        for x, lo, w in streams:
            emb = x + tt * td_ref[0][None, None, lo:lo + w]
            embs.append((emb, lo, w))
            s = s + jnp.sum(emb, axis=-1, keepdims=True)
            sq = sq + jnp.sum(emb * emb, axis=-1, keepdims=True)
        mu = s * (1.0 / hd)
        var = sq * (1.0 / hd) - mu * mu
        rstd = lax.rsqrt(var + EPS)
        mr = mu * rstd
        for emb, lo, w in embs:
            g = g_ref[0][None, None, lo:lo + w]
            b = b_ref[0][None, None, lo:lo + w]
            out_ref[:, :, lo:lo + w] = (emb * rstd - mr) * g + b

    def body(*refs):
        we_ref, tt_ref, am_ref, pe_ref, td_ref, g_ref, b_ref = refs[:7]
        out_ref, mask_ref = refs[-2], refs[-1]
        tt = tt_ref[...].astype(jnp.float32)[..., None]
        if packed:
            x = we_ref[...]
            lo = lax.bitcast_convert_type(x << 16, jnp.float32) + pe_ref[:, :Hh][None]
            hi = (lax.bitcast_convert_type(x & jnp.int32(-65536), jnp.float32)
                  + pe_ref[:, Hh:][None])
            streams = [(lo, 0, Hh), (hi, Hh, Hh)]
        else:
            streams = [(we_ref[...] + pe_ref[...][None], 0, hd)]
        _ln_store(streams, tt, td_ref, g_ref, b_ref, out_ref)
        amf = am_ref[...].astype(jnp.float32)
        mask_ref[...] = ((1.0 - amf) * -10000.0)[:, None, :]

    in_specs = [
        pl.BlockSpec((BB, S, W), lambda i: (i, 0, 0)),
        pl.BlockSpec((BB, S), lambda i: (i, 0)),
        pl.BlockSpec((BB, S), lambda i: (i, 0)),
        pl.BlockSpec((S, hd), lambda i: (0, 0)),
        pl.BlockSpec((1, hd), lambda i: (0, 0)),
        pl.BlockSpec((1, hd), lambda i: (0, 0)),
        pl.BlockSpec((1, hd), lambda i: (0, 0)),
    ]
    args = [we3, token_type_ids, am, pe2, tokd2, gamma2, beta2]
    aliases = {}
    if slice_idx > 0:
        in_specs.append(pl.BlockSpec(memory_space=pl.ANY))
        in_specs.append(pl.BlockSpec(memory_space=pl.ANY))
        args.append(prev_out)
        args.append(prev_mask)
        aliases = {7: 0, 8: 1}

    return pl.pallas_call(
        body,
        grid=grid,
        in_specs=in_specs,
        out_specs=[
            pl.BlockSpec((BB, S, hd), lambda i: (i + off, 0, 0)),
            pl.BlockSpec((BB, 1, S), lambda i: (i + off, 0, 0)),
        ],
        out_shape=[
            jax.ShapeDtypeStruct((b_total, S, hd), jnp.float32),
            jax.ShapeDtypeStruct((b_total, 1, S), jnp.float32),
        ],
        input_output_aliases=aliases,
    )(*args)


def _pack_table(word_emb):
    V, Hd = word_emb.shape
    Hh = Hd // 2
    R = 2048
    grid = (pl.cdiv(V, R),)

    def body(w_ref, o_ref):
        x = w_ref[...]
        lob = x[:, :Hh].astype(jnp.bfloat16)
        hib = x[:, Hh:].astype(jnp.bfloat16)
        lo16 = lax.bitcast_convert_type(lob, jnp.uint16).astype(jnp.int32)
        hi16 = lax.bitcast_convert_type(hib, jnp.uint16).astype(jnp.int32)
        o_ref[...] = lo16 | (hi16 << 16)

    return pl.pallas_call(
        body,
        grid=grid,
        in_specs=[pl.BlockSpec((R, Hd), lambda i: (i, 0))],
        out_specs=pl.BlockSpec((R, Hh), lambda i: (i, 0)),
        out_shape=jax.ShapeDtypeStruct((V, Hh), jnp.int32),
    )(word_emb)


def kernel(input_ids, attention_mask, token_type_ids, word_emb, pos_emb, tok_emb, gamma, beta):
    B, S = input_ids.shape
    V, Hd = word_emb.shape
    n = B * S
    ids = input_ids.reshape(-1).astype(jnp.int32)
    SLICES = [256, 256, 256, 256]
    tt = token_type_ids.astype(jnp.int32)
    pe2 = pos_emb[:S] + tok_emb[0][None, :]
    tokd2 = (tok_emb[1] - tok_emb[0]).reshape(1, Hd)
    gamma2 = gamma.reshape(1, Hd)
    beta2 = beta.reshape(1, Hd)
    Hh = Hd // 2
    offs = [sum(SLICES[:i]) for i in range(len(SLICES))]
    word_emb_h = _pack_table(word_emb)
    we_slices = [
        _sc_gather(word_emb_h, ids[b0 * S:(b0 + bs) * S], bs * S, Hh, jnp.int32)
        .reshape(bs, S, Hh)
        for b0, bs in zip(offs, SLICES)
    ]
    am = attention_mask.astype(jnp.int32)
    out = None
    mask = None
    for i, (b0, bs, we) in enumerate(zip(offs, SLICES, we_slices)):
        out, mask = _tc_fused_slice(we,
                                    tt[b0:b0 + bs],
                                    am[b0:b0 + bs],
                                    pe2, tokd2, gamma2, beta2,
                                    out, mask, i, B, Hd, b0, packed=True)
    return (out, mask)
